# initial kernel scaffold (unmeasured)
import jax
import jax.numpy as jnp
from jax import lax
from jax.experimental import pallas as pl
from jax.experimental.pallas import tpu as pltpu

N_DEV = 4
EPS = 1e-5
BLOCK_M = 1024


def _rrms_body(x_ref, out_ref, acc_ref, comm_ref, send_sems, recv_sems):
    i = pl.program_id(0)
    nsteps = pl.num_programs(0)
    me = lax.axis_index("i")

    xf = x_ref[...].astype(jnp.float32)
    acc_ref[pl.ds(i * BLOCK_M, BLOCK_M), :] = jnp.sum(
        xf * xf, axis=1, keepdims=True
    )

    @pl.when(i == nsteps - 1)
    def _():
        barrier = pltpu.get_barrier_semaphore()
        for d in range(1, N_DEV):
            peer = (me + d) % N_DEV
            pl.semaphore_signal(
                barrier, inc=1,
                device_id=(peer,), device_id_type=pl.DeviceIdType.MESH,
            )
        pl.semaphore_wait(barrier, N_DEV - 1)

        sends = []
        for d in range(1, N_DEV):
            peer = (me + d) % N_DEV
            rdma = pltpu.make_async_remote_copy(
                src_ref=acc_ref,
                dst_ref=comm_ref.at[me],
                send_sem=send_sems.at[d - 1],
                recv_sem=recv_sems.at[me],
                device_id=(peer,),
                device_id_type=pl.DeviceIdType.MESH,
            )
            rdma.start()
            sends.append(rdma)

        for d in range(1, N_DEV):
            s = (me - d + N_DEV) % N_DEV
            recv = pltpu.make_async_remote_copy(
                src_ref=acc_ref,
                dst_ref=comm_ref.at[s],
                send_sem=send_sems.at[0],
                recv_sem=recv_sems.at[s],
                device_id=(me,),
                device_id_type=pl.DeviceIdType.MESH,
            )
            recv.wait_recv()

        total = acc_ref[...]
        for s in range(N_DEV):
            total = total + jnp.where(me == s, 0.0, comm_ref[s, :, :])

        for rdma in sends:
            rdma.wait_send()

        out_ref[...] = lax.rsqrt(total * (1.0 / (N_DEV * 2048.0)) + EPS)


def _scale_body(x_ref, r_ref, g_ref, out_ref):
    xf = x_ref[...].astype(jnp.float32)
    out_ref[...] = (xf * r_ref[...] * g_ref[...]).astype(out_ref.dtype)


def kernel(x, gamma):
    m, n_loc = x.shape
    nsteps = m // BLOCK_M

    rrms = pl.pallas_call(
        _rrms_body,
        grid=(nsteps,),
        out_shape=jax.ShapeDtypeStruct((m, 1), jnp.float32),
        in_specs=[
            pl.BlockSpec((BLOCK_M, n_loc), lambda i: (i, 0),
                         memory_space=pltpu.VMEM),
        ],
        out_specs=pl.BlockSpec((m, 1), lambda i: (0, 0),
                               memory_space=pltpu.VMEM),
        scratch_shapes=[
            pltpu.VMEM((m, 1), jnp.float32),
            pltpu.VMEM((N_DEV, m, 1), jnp.float32),
            pltpu.SemaphoreType.DMA((N_DEV - 1,)),
            pltpu.SemaphoreType.DMA((N_DEV,)),
        ],
        compiler_params=pltpu.CompilerParams(collective_id=0),
    )(x)

    g2 = gamma.reshape(1, n_loc)

    out = pl.pallas_call(
        _scale_body,
        grid=(nsteps,),
        out_shape=jax.ShapeDtypeStruct((m, n_loc), jnp.bfloat16),
        in_specs=[
            pl.BlockSpec((BLOCK_M, n_loc), lambda i: (i, 0),
                         memory_space=pltpu.VMEM),
            pl.BlockSpec((BLOCK_M, 1), lambda i: (i, 0),
                         memory_space=pltpu.VMEM),
            pl.BlockSpec((1, n_loc), lambda i: (0, 0),
                         memory_space=pltpu.VMEM),
        ],
        out_specs=pl.BlockSpec((BLOCK_M, n_loc), lambda i: (i, 0),
                               memory_space=pltpu.VMEM),
    )(x, rrms, g2)
    return out


# baseline (device time: 155236 ns/iter reference)
import jax
import jax.numpy as jnp
from jax import lax
from jax.experimental import pallas as pl
from jax.experimental.pallas import tpu as pltpu

N_DEV = 4
EPS = 1e-5
BLOCK_M = 1024


def _rrms_body(x_ref, out_ref, acc_ref, comm_ref, send_sems, recv_sems):
    i = pl.program_id(0)
    nsteps = pl.num_programs(0)
    me = lax.axis_index("i")

    xf = x_ref[...].astype(jnp.float32)
    acc_ref[pl.ds(i * BLOCK_M, BLOCK_M), :] = jnp.sum(
        xf * xf, axis=1, keepdims=True
    )

    @pl.when(i == nsteps - 1)
    def _():
        barrier = pltpu.get_barrier_semaphore()
        for d in range(1, N_DEV):
            peer = (me + d) % N_DEV
            pl.semaphore_signal(
                barrier, inc=1,
                device_id=(peer,), device_id_type=pl.DeviceIdType.MESH,
            )
        pl.semaphore_wait(barrier, N_DEV - 1)

        sends = []
        for d in range(1, N_DEV):
            peer = (me + d) % N_DEV
            rdma = pltpu.make_async_remote_copy(
                src_ref=acc_ref,
                dst_ref=comm_ref.at[me],
                send_sem=send_sems.at[d - 1],
                recv_sem=recv_sems.at[me],
                device_id=(peer,),
                device_id_type=pl.DeviceIdType.MESH,
            )
            rdma.start()
            sends.append(rdma)

        for d in range(1, N_DEV):
            s = (me - d + N_DEV) % N_DEV
            recv = pltpu.make_async_remote_copy(
                src_ref=acc_ref,
                dst_ref=comm_ref.at[s],
                send_sem=send_sems.at[0],
                recv_sem=recv_sems.at[s],
                device_id=(me,),
                device_id_type=pl.DeviceIdType.MESH,
            )
            recv.wait_recv()

        total = acc_ref[...]
        for s in range(N_DEV):
            total = total + jnp.where(me == s, 0.0, comm_ref[s, :, :])

        for rdma in sends:
            rdma.wait_send()

        out_ref[...] = lax.rsqrt(total * (1.0 / (N_DEV * 2048.0)) + EPS)


def _scale_body(x_ref, r_ref, g_ref, out_ref):
    xf = x_ref[...].astype(jnp.float32)
    out_ref[...] = (xf * r_ref[...] * g_ref[...]).astype(out_ref.dtype)


def kernel(x, gamma):
    m, n_loc = x.shape
    nsteps = m // BLOCK_M

    rrms = pl.pallas_call(
        _rrms_body,
        grid=(nsteps,),
        out_shape=jax.ShapeDtypeStruct((m, 1), jnp.float32),
        in_specs=[
            pl.BlockSpec((BLOCK_M, n_loc), lambda i: (i, 0),
                         memory_space=pltpu.VMEM),
        ],
        out_specs=pl.BlockSpec((m, 1), lambda i: (0, 0),
                               memory_space=pltpu.VMEM),
        scratch_shapes=[
            pltpu.VMEM((m, 1), jnp.float32),
            pltpu.VMEM((N_DEV, m, 1), jnp.float32),
            pltpu.SemaphoreType.DMA((N_DEV - 1,)),
            pltpu.SemaphoreType.DMA((N_DEV,)),
        ],
        compiler_params=pltpu.CompilerParams(
            collective_id=0, vmem_limit_bytes=56 * 1024 * 1024
        ),
    )(x)

    g2 = gamma.reshape(1, n_loc)

    out = pl.pallas_call(
        _scale_body,
        grid=(nsteps,),
        out_shape=jax.ShapeDtypeStruct((m, n_loc), jnp.bfloat16),
        in_specs=[
            pl.BlockSpec((BLOCK_M, n_loc), lambda i: (i, 0),
                         memory_space=pltpu.VMEM),
            pl.BlockSpec((BLOCK_M, 1), lambda i: (i, 0),
                         memory_space=pltpu.VMEM),
            pl.BlockSpec((1, n_loc), lambda i: (0, 0),
                         memory_space=pltpu.VMEM),
        ],
        out_specs=pl.BlockSpec((BLOCK_M, n_loc), lambda i: (i, 0),
                               memory_space=pltpu.VMEM),
    )(x, rrms, g2)
    return out


# device time: 65259 ns/iter; 2.3788x vs baseline; 2.3788x over previous
import jax
import jax.numpy as jnp
from jax import lax
from jax.experimental import pallas as pl
from jax.experimental.pallas import tpu as pltpu

N_DEV = 4
EPS = 1e-5
BLOCK_M = 1024
LANES = 128


def _pack_rows(s, nrows):
    r_idx = lax.broadcasted_iota(jnp.int32, (nrows, LANES), 0)
    b_idx = lax.broadcasted_iota(jnp.int32, (nrows, LANES), 1)
    masked = s * (r_idx % LANES == b_idx).astype(jnp.float32)
    a_idx = lax.broadcasted_iota(jnp.int32, (nrows // LANES, nrows), 0)
    rr_idx = lax.broadcasted_iota(jnp.int32, (nrows // LANES, nrows), 1)
    sel = (rr_idx // LANES == a_idx).astype(jnp.float32)
    return jax.lax.dot(sel, masked, preferred_element_type=jnp.float32)


def _unpack_rows(p, nrows):
    r_idx = lax.broadcasted_iota(jnp.int32, (nrows, LANES), 0)
    b_idx = lax.broadcasted_iota(jnp.int32, (nrows, LANES), 1)
    a_idx = lax.broadcasted_iota(jnp.int32, (nrows, nrows // LANES), 1)
    rr_idx = lax.broadcasted_iota(jnp.int32, (nrows, nrows // LANES), 0)
    sel = (rr_idx // LANES == a_idx).astype(jnp.float32)
    w = jax.lax.dot(sel, p, preferred_element_type=jnp.float32)
    w = w * (r_idx % LANES == b_idx).astype(jnp.float32)
    return jnp.sum(w, axis=1, keepdims=True)


def _rrms_body(x_ref, out_ref, acc_ref, comm_ref, send_sems, recv_sems):
    i = pl.program_id(0)
    nsteps = pl.num_programs(0)
    me = lax.axis_index("i")
    m = out_ref.shape[0]

    xf = x_ref[...]
    s = jnp.sum(xf * xf, axis=1, keepdims=True)
    packed = _pack_rows(s, BLOCK_M)
    acc_ref[pl.ds(i * (BLOCK_M // LANES), BLOCK_M // LANES), :] = packed

    @pl.when(i == nsteps - 1)
    def _():
        barrier = pltpu.get_barrier_semaphore()
        for d in range(1, N_DEV):
            peer = (me + d) % N_DEV
            pl.semaphore_signal(
                barrier, inc=1,
                device_id=(peer,), device_id_type=pl.DeviceIdType.MESH,
            )
        pl.semaphore_wait(barrier, N_DEV - 1)

        sends = []
        for d in range(1, N_DEV):
            peer = (me + d) % N_DEV
            rdma = pltpu.make_async_remote_copy(
                src_ref=acc_ref,
                dst_ref=comm_ref.at[me],
                send_sem=send_sems.at[d - 1],
                recv_sem=recv_sems.at[me],
                device_id=(peer,),
                device_id_type=pl.DeviceIdType.MESH,
            )
            rdma.start()
            sends.append(rdma)

        for d in range(1, N_DEV):
            src = (me - d + N_DEV) % N_DEV
            recv = pltpu.make_async_remote_copy(
                src_ref=acc_ref,
                dst_ref=comm_ref.at[src],
                send_sem=send_sems.at[0],
                recv_sem=recv_sems.at[src],
                device_id=(me,),
                device_id_type=pl.DeviceIdType.MESH,
            )
            recv.wait_recv()

        total = acc_ref[...]
        for peer in range(N_DEV):
            total = total + jnp.where(me == peer, 0.0, comm_ref[peer, :, :])

        for rdma in sends:
            rdma.wait_send()

        rrms_packed = lax.rsqrt(total * (1.0 / (N_DEV * 2048.0)) + EPS)
        out_ref[...] = _unpack_rows(rrms_packed, m)


def _scale_body(x_ref, r_ref, g_ref, out_ref):
    xf = x_ref[...]
    out_ref[...] = (xf * r_ref[...] * g_ref[...]).astype(out_ref.dtype)


def kernel(x, gamma):
    m, n_loc = x.shape
    nsteps = m // BLOCK_M

    rrms = pl.pallas_call(
        _rrms_body,
        grid=(nsteps,),
        out_shape=jax.ShapeDtypeStruct((m, 1), jnp.float32),
        in_specs=[
            pl.BlockSpec((BLOCK_M, n_loc), lambda i: (i, 0),
                         memory_space=pltpu.VMEM),
        ],
        out_specs=pl.BlockSpec((m, 1), lambda i: (0, 0),
                               memory_space=pltpu.VMEM),
        scratch_shapes=[
            pltpu.VMEM((m // LANES, LANES), jnp.float32),
            pltpu.VMEM((N_DEV, m // LANES, LANES), jnp.float32),
            pltpu.SemaphoreType.DMA((N_DEV - 1,)),
            pltpu.SemaphoreType.DMA((N_DEV,)),
        ],
        compiler_params=pltpu.CompilerParams(
            collective_id=0, vmem_limit_bytes=56 * 1024 * 1024
        ),
    )(x)

    g2 = gamma.reshape(1, n_loc)

    out = pl.pallas_call(
        _scale_body,
        grid=(nsteps,),
        out_shape=jax.ShapeDtypeStruct((m, n_loc), jnp.bfloat16),
        in_specs=[
            pl.BlockSpec((BLOCK_M, n_loc), lambda i: (i, 0),
                         memory_space=pltpu.VMEM),
            pl.BlockSpec((BLOCK_M, 1), lambda i: (i, 0),
                         memory_space=pltpu.VMEM),
            pl.BlockSpec((1, n_loc), lambda i: (0, 0),
                         memory_space=pltpu.VMEM),
        ],
        out_specs=pl.BlockSpec((BLOCK_M, n_loc), lambda i: (i, 0),
                               memory_space=pltpu.VMEM),
    )(x, rrms, g2)
    return out
